# baseline (speedup 1.0000x reference)
import functools

import jax
import jax.numpy as jnp
from jax import lax
from jax.experimental import pallas as pl
from jax.experimental.pallas import tpu as pltpu
from jax.experimental.pallas import tpu_sc as plsc

NC = 2
NS = 16
NW = NC * NS
L = 16

_SC_PARAMS = pltpu.CompilerParams(
    needs_layout_passes=False, use_tc_tiling_on_sc=False)


def _iota16():
    return lax.iota(jnp.int32, L)


def _full16(v):
    return jnp.full((L,), v, jnp.int32)


def _nchunks(n, b):
    return -(-n // b)


def _make_fused(n_pts, n_elem, n_nodes, b1, b2):
    chunks1 = _nchunks(n_elem, b1)
    last1 = n_elem - b1
    chunks2 = _nchunks(n_pts, b2)
    last2 = n_pts - b2
    mesh = plsc.VectorSubcoreMesh(core_axis_name="c", subcore_axis_name="s")
    out_sds = jax.ShapeDtypeStruct((n_pts,), jnp.float32)
    m_sds = jax.ShapeDtypeStruct((n_elem, 9), jnp.float32)

    @functools.partial(
        pl.kernel,
        out_type=(out_sds, out_sds, out_sds, m_sds),
        mesh=mesh,
        scratch_types=[
            pltpu.VMEM((b1,), jnp.int32),
            pltpu.VMEM((b1,), jnp.int32),
            pltpu.VMEM((b1,), jnp.int32),
            pltpu.VMEM((b1, 2), jnp.float32),
            pltpu.VMEM((b1, 2), jnp.float32),
            pltpu.VMEM((b1, 2), jnp.float32),
            pltpu.VMEM((b1, 9), jnp.float32),
            pltpu.VMEM((b2,), jnp.int32),
            pltpu.VMEM((b2,), jnp.float32),
            pltpu.VMEM((b2,), jnp.float32),
            pltpu.VMEM((b2, 9), jnp.float32),
            pltpu.VMEM((b2,), jnp.float32),
            pltpu.VMEM((b2,), jnp.float32),
            pltpu.VMEM((b2,), jnp.float32),
            pltpu.SemaphoreType.DMA,
            pltpu.SemaphoreType.DMA,
            pltpu.SemaphoreType.DMA,
        ],
        compiler_params=_SC_PARAMS,
    )
    def fused(cid_hbm, xt_hbm, conn_hbm, coords_hbm,
              o0_hbm, o1_hbm, o2_hbm, m_hbm,
              i1_v, i2_v, i3_v, c1_v, c2_v, c3_v, m_v,
              cid_v, x_v, y_v, mg_v, o0_v, o1_v, o2_v,
              sem1, sem2, sem3):
        sc = lax.axis_index("c")
        tid = lax.axis_index("s")
        wid = tid * NC + sc

        @pl.loop(tid, chunks1, step=NS)
        def _chunk1(c):
            base = jnp.minimum(c * b1, last1)
            pltpu.sync_copy(conn_hbm.at[pl.ds(base, b1)], i1_v)
            pltpu.sync_copy(conn_hbm.at[pl.ds(n_elem + base, b1)], i2_v)
            pltpu.sync_copy(conn_hbm.at[pl.ds(2 * n_elem + base, b1)], i3_v)

            @pl.loop(0, b1 // L)
            def _idx(i):
                s = pl.ds(i * L, L)
                i1_v[s] = jnp.maximum(i1_v[s] - 1, 0)
                i2_v[s] = jnp.maximum(i2_v[s] - 1, 0)
                i3_v[s] = jnp.maximum(i3_v[s] - 1, 0)

            d1_ = pltpu.async_copy(coords_hbm.at[i1_v], c1_v, sem1)
            d2_ = pltpu.async_copy(coords_hbm.at[i2_v], c2_v, sem2)
            d3_ = pltpu.async_copy(coords_hbm.at[i3_v], c3_v, sem3)
            d1_.wait()
            d2_.wait()
            d3_.wait()

            @pl.loop(0, b1 // L)
            def _mat(i):
                rows = i * L + _iota16()
                z = _full16(0)
                o = _full16(1)
                x1 = plsc.load_gather(c1_v, [rows, z])
                y1 = plsc.load_gather(c1_v, [rows, o])
                x2 = plsc.load_gather(c2_v, [rows, z])
                y2 = plsc.load_gather(c2_v, [rows, o])
                x3 = plsc.load_gather(c3_v, [rows, z])
                y3 = plsc.load_gather(c3_v, [rows, o])
                d1 = x1 * (y3 - y2) + x2 * (y1 - y3) + x3 * (y2 - y1)
                d2 = (-x1 * y2 + x1 * y3 + x2 * y1 - x2 * y3
                      - x3 * y1 + x3 * y2)
                d3 = x1 * (y2 - y3) + x2 * (y3 - y1) + x3 * (y1 - y2)
                vals = (
                    (y3 - y2) / d1,
                    (x2 - x3) / d2,
                    (x3 * y2 - x2 * y3) / d2,
                    (y1 - y3) / d2,
                    (x1 - x3) / d3,
                    (x3 * y1 - x1 * y3) / d3,
                    (y1 - y2) / d3,
                    (x1 - x2) / d2,
                    (x2 * y1 - x1 * y2) / d2,
                )
                for col, val in enumerate(vals):
                    plsc.store_scatter(m_v, [rows, _full16(col)], val)

            pltpu.sync_copy(m_v, m_hbm.at[pl.ds(base, b1)])

        plsc.subcore_barrier()

        @pl.loop(wid, chunks2, step=NW)
        def _chunk2(c):
            base = jnp.minimum(c * b2, last2)
            dx = pltpu.async_copy(xt_hbm.at[pl.ds(base, b2)], x_v, sem2)
            dy = pltpu.async_copy(xt_hbm.at[pl.ds(n_pts + base, b2)], y_v,
                                  sem3)
            pltpu.sync_copy(cid_hbm.at[pl.ds(base, b2)], cid_v)
            pltpu.async_copy(m_hbm.at[cid_v], mg_v, sem1).wait()
            dx.wait()
            dy.wait()

            @pl.loop(0, b2 // L)
            def _pt(i):
                s = pl.ds(i * L, L)
                rows = i * L + _iota16()
                x = x_v[s]
                y = y_v[s]
                m = [plsc.load_gather(mg_v, [rows, _full16(col)])
                     for col in range(9)]
                o0_v[s] = x * m[0] + y * m[1] + m[2]
                o1_v[s] = x * m[3] + y * m[4] + m[5]
                o2_v[s] = x * m[6] + y * m[7] + m[8]

            pltpu.sync_copy(o0_v, o0_hbm.at[pl.ds(base, b2)])
            pltpu.sync_copy(o1_v, o1_hbm.at[pl.ds(base, b2)])
            pltpu.sync_copy(o2_v, o2_hbm.at[pl.ds(base, b2)])

    return fused


def kernel(x, cell_id, coordinates, nodal_values, connectivity):
    del nodal_values
    n_pts = x.shape[0]
    n_elem = connectivity.shape[0]
    n_nodes = coordinates.shape[0]

    b1 = 896
    b2 = 1120

    coords2 = coordinates.reshape(n_nodes, 2)
    conn_t = connectivity.T.reshape(3 * n_elem)
    xt = x.T.reshape(2 * n_pts)

    o0, o1, o2, _ = _make_fused(n_pts, n_elem, n_nodes, b1, b2)(
        cell_id, xt, conn_t, coords2)
    return jnp.stack([o0, o1, o2], axis=1)


# speedup vs baseline: 1.4803x; 1.4803x over previous
import functools

import jax
import jax.numpy as jnp
from jax import lax
from jax.experimental import pallas as pl
from jax.experimental.pallas import tpu as pltpu
from jax.experimental.pallas import tpu_sc as plsc

NC = 2
NS = 16
NW = NC * NS
L = 16

B1 = 1280
K1 = 5
B2 = 1600
K2 = 10

_SC_PARAMS = pltpu.CompilerParams(
    needs_layout_passes=False, use_tc_tiling_on_sc=False)


def _iota16():
    return lax.iota(jnp.int32, L)


def _full16(v):
    return jnp.full((L,), v, jnp.int32)


def _make_phase1(n_elem, n_nodes):
    last1 = n_elem - B1
    mesh = plsc.VectorSubcoreMesh(core_axis_name="c", subcore_axis_name="s")
    vset = [
        pltpu.VMEM((B1,), jnp.int32),
        pltpu.VMEM((B1,), jnp.int32),
        pltpu.VMEM((B1,), jnp.int32),
        pltpu.VMEM((B1, 2), jnp.float32),
        pltpu.VMEM((B1, 2), jnp.float32),
        pltpu.VMEM((B1, 2), jnp.float32),
        pltpu.VMEM((B1, 9), jnp.float32),
        pltpu.SemaphoreType.DMA,
        pltpu.SemaphoreType.DMA,
        pltpu.SemaphoreType.DMA,
    ]

    @functools.partial(
        pl.kernel,
        out_type=jax.ShapeDtypeStruct((n_elem, 9), jnp.float32),
        mesh=mesh,
        scratch_types=vset + vset,
        compiler_params=_SC_PARAMS,
    )
    def phase1(conn_hbm, coords_hbm, m_hbm, *bufs):
        sets = [bufs[:10], bufs[10:]]
        wid = lax.axis_index("s") * NC + lax.axis_index("c")
        first = wid * K1

        def base_of(j):
            return jnp.minimum((first + j) * B1, last1)

        def issue_conn(j, st):
            i1_v, i2_v, i3_v = st[0], st[1], st[2]
            base = base_of(j)
            a = pltpu.async_copy(conn_hbm.at[pl.ds(base, B1)], i1_v, st[7])
            b = pltpu.async_copy(
                conn_hbm.at[pl.ds(n_elem + base, B1)], i2_v, st[7])
            c = pltpu.async_copy(
                conn_hbm.at[pl.ds(2 * n_elem + base, B1)], i3_v, st[7])
            return (a, b, c)

        def idx_and_gather(st):
            i1_v, i2_v, i3_v = st[0], st[1], st[2]

            @pl.loop(0, B1 // L)
            def _idx(i):
                s = pl.ds(i * L, L)
                i1_v[s] = jnp.maximum(i1_v[s] - 1, 0)
                i2_v[s] = jnp.maximum(i2_v[s] - 1, 0)
                i3_v[s] = jnp.maximum(i3_v[s] - 1, 0)

            a = pltpu.async_copy(coords_hbm.at[i1_v], st[3], st[8])
            b = pltpu.async_copy(coords_hbm.at[i2_v], st[4], st[8])
            c = pltpu.async_copy(coords_hbm.at[i3_v], st[5], st[8])
            return (a, b, c)

        def mat_and_store(j, st):
            c1_v, c2_v, c3_v, m_v = st[3], st[4], st[5], st[6]

            @pl.loop(0, B1 // L)
            def _mat(i):
                rows = i * L + _iota16()
                z = _full16(0)
                o = _full16(1)
                x1 = plsc.load_gather(c1_v, [rows, z])
                y1 = plsc.load_gather(c1_v, [rows, o])
                x2 = plsc.load_gather(c2_v, [rows, z])
                y2 = plsc.load_gather(c2_v, [rows, o])
                x3 = plsc.load_gather(c3_v, [rows, z])
                y3 = plsc.load_gather(c3_v, [rows, o])
                d1 = x1 * (y3 - y2) + x2 * (y1 - y3) + x3 * (y2 - y1)
                d2 = (-x1 * y2 + x1 * y3 + x2 * y1 - x2 * y3
                      - x3 * y1 + x3 * y2)
                d3 = x1 * (y2 - y3) + x2 * (y3 - y1) + x3 * (y1 - y2)
                vals = (
                    (y3 - y2) / d1,
                    (x2 - x3) / d2,
                    (x3 * y2 - x2 * y3) / d2,
                    (y1 - y3) / d2,
                    (x1 - x3) / d3,
                    (x3 * y1 - x1 * y3) / d3,
                    (y1 - y2) / d3,
                    (x1 - x2) / d2,
                    (x2 * y1 - x1 * y2) / d2,
                )
                for col, val in enumerate(vals):
                    plsc.store_scatter(m_v, [rows, _full16(col)], val)

            return pltpu.async_copy(
                m_v, m_hbm.at[pl.ds(base_of(j), B1)], st[9])

        conn_d = {0: issue_conn(0, sets[0])}
        gath_d = {}
        store_d = {}
        for j in range(K1):
            cur = sets[j % 2]
            prv = sets[(j + 1) % 2]
            for d in conn_d.pop(j):
                d.wait()
            if j >= 2:
                store_d.pop(j - 2).wait()
            gath_d[j] = idx_and_gather(cur)
            if j >= 1:
                for d in gath_d.pop(j - 1):
                    d.wait()
                if j + 1 < K1:
                    conn_d[j + 1] = issue_conn(j + 1, prv)
                store_d[j - 1] = mat_and_store(j - 1, prv)
            elif j + 1 < K1:
                conn_d[j + 1] = issue_conn(j + 1, prv)
        for d in gath_d.pop(K1 - 1):
            d.wait()
        store_d[K1 - 1] = mat_and_store(K1 - 1, sets[(K1 - 1) % 2])
        for j in sorted(store_d):
            store_d.pop(j).wait()

    return phase1


def _make_phase2(n_pts, n_elem):
    last2 = n_pts - B2
    mesh = plsc.VectorSubcoreMesh(core_axis_name="c", subcore_axis_name="s")
    out_sds = jax.ShapeDtypeStruct((n_pts,), jnp.float32)
    vset = [
        pltpu.VMEM((B2,), jnp.int32),
        pltpu.VMEM((B2,), jnp.float32),
        pltpu.VMEM((B2,), jnp.float32),
        pltpu.VMEM((B2, 9), jnp.float32),
        pltpu.VMEM((B2,), jnp.float32),
        pltpu.VMEM((B2,), jnp.float32),
        pltpu.VMEM((B2,), jnp.float32),
        pltpu.SemaphoreType.DMA,
        pltpu.SemaphoreType.DMA,
        pltpu.SemaphoreType.DMA,
    ]

    @functools.partial(
        pl.kernel,
        out_type=(out_sds, out_sds, out_sds),
        mesh=mesh,
        scratch_types=vset + vset,
        compiler_params=_SC_PARAMS,
    )
    def phase2(cid_hbm, xt_hbm, m_hbm, o0_hbm, o1_hbm, o2_hbm, *bufs):
        sets = [bufs[:10], bufs[10:]]
        wid = lax.axis_index("s") * NC + lax.axis_index("c")
        first = wid * K2

        def base_of(j):
            return jnp.minimum((first + j) * B2, last2)

        def issue_cid(j, st):
            base = base_of(j)
            return pltpu.async_copy(cid_hbm.at[pl.ds(base, B2)], st[0],
                                    st[7])

        def issue_xy(j, st):
            base = base_of(j)
            a = pltpu.async_copy(xt_hbm.at[pl.ds(base, B2)], st[1], st[7])
            b = pltpu.async_copy(
                xt_hbm.at[pl.ds(n_pts + base, B2)], st[2], st[7])
            return (a, b)

        def issue_gather(st):
            return pltpu.async_copy(m_hbm.at[st[0]], st[3], st[8])

        def compute_and_store(j, st):
            x_v, y_v, mg_v = st[1], st[2], st[3]
            o0_v, o1_v, o2_v = st[4], st[5], st[6]

            @pl.loop(0, B2 // L)
            def _pt(i):
                s = pl.ds(i * L, L)
                rows = i * L + _iota16()
                x = x_v[s]
                y = y_v[s]
                m = [plsc.load_gather(mg_v, [rows, _full16(col)])
                     for col in range(9)]
                o0_v[s] = x * m[0] + y * m[1] + m[2]
                o1_v[s] = x * m[3] + y * m[4] + m[5]
                o2_v[s] = x * m[6] + y * m[7] + m[8]

            base = base_of(j)
            a = pltpu.async_copy(o0_v, o0_hbm.at[pl.ds(base, B2)], st[9])
            b = pltpu.async_copy(o1_v, o1_hbm.at[pl.ds(base, B2)], st[9])
            c = pltpu.async_copy(o2_v, o2_hbm.at[pl.ds(base, B2)], st[9])
            return (a, b, c)

        cid_d = {0: issue_cid(0, sets[0]), 1: issue_cid(1, sets[1])}
        xy_d = {0: issue_xy(0, sets[0]), 1: issue_xy(1, sets[1])}
        gath_d = {}
        store_d = {}
        for j in range(K2):
            cur = sets[j % 2]
            prv = sets[(j + 1) % 2]
            cid_d.pop(j).wait()
            gath_d[j] = issue_gather(cur)
            if j >= 1:
                if j + 1 < K2:
                    pass
                gath_d.pop(j - 1).wait()
                xa, ya = xy_d.pop(j - 1)
                xa.wait()
                ya.wait()
                if j + 1 < K2:
                    cid_d[j + 1] = issue_cid(j + 1, prv)
                if j >= 2:
                    for d in store_d.pop(j - 2):
                        d.wait()
                store_d[j - 1] = compute_and_store(j - 1, prv)
                if j + 1 < K2:
                    xy_d[j + 1] = issue_xy(j + 1, prv)
        gath_d.pop(K2 - 1).wait()
        xa, ya = xy_d.pop(K2 - 1)
        xa.wait()
        ya.wait()
        store_d[K2 - 1] = compute_and_store(K2 - 1, sets[(K2 - 1) % 2])
        for j in sorted(store_d):
            for d in store_d.pop(j):
                d.wait()

    return phase2


def kernel(x, cell_id, coordinates, nodal_values, connectivity):
    del nodal_values
    n_pts = x.shape[0]
    n_elem = connectivity.shape[0]
    n_nodes = coordinates.shape[0]

    coords2 = coordinates.reshape(n_nodes, 2)
    conn_t = connectivity.T.reshape(3 * n_elem)
    xt = x.T.reshape(2 * n_pts)

    m_table = _make_phase1(n_elem, n_nodes)(conn_t, coords2)
    o0, o1, o2 = _make_phase2(n_pts, n_elem)(cell_id, xt, m_table)
    return jnp.stack([o0, o1, o2], axis=1)
